# Initial kernel scaffold; baseline (speedup 1.0000x reference)
import jax
import jax.numpy as jnp
from jax.experimental import pallas as pl


def kernel(x, edge_index, W1, b1, W2, b2, W3, b3):
    raise NotImplementedError("write your pallas kernel here")


# speedup vs baseline: 16.7651x; 16.7651x over previous
import functools

import jax
import jax.numpy as jnp
from jax import lax
from jax.experimental import pallas as pl
from jax.experimental.pallas import tpu as pltpu
from jax.experimental.pallas import tpu_sc as plsc

NC = 2
NS = 16
NW = NC * NS
C = 128


@functools.lru_cache(maxsize=None)
def _make_scatter(n_pad: int, nchunk: int, d: int):
    rows_pt = n_pad // NS
    assert rows_pt % C == 0
    nzc = rows_pt // C
    mesh = plsc.VectorSubcoreMesh(
        core_axis_name="c", subcore_axis_name="s",
        num_cores=NC, num_subcores=NS)

    @functools.partial(
        pl.kernel,
        out_type=jax.ShapeDtypeStruct((NC, n_pad, d), jnp.float32),
        mesh=mesh,
        scratch_types=[
            pltpu.VMEM((nchunk, C), jnp.int32),
            pltpu.VMEM((nchunk, C), jnp.int32),
            pltpu.VMEM((C, d), jnp.float32),
            pltpu.VMEM((C, d), jnp.float32),
            pltpu.VMEM_SHARED((n_pad, d), jnp.float32),
        ],
        compiler_params=pltpu.CompilerParams(use_tc_tiling_on_sc=False),
    )
    def scat(g_hbm, srci_hbm, dsti_hbm, zrow_hbm, out_hbm,
             srci, dsti, msg, stage, acc):
        cid = lax.axis_index("c")
        sid = lax.axis_index("s")
        wid = sid * NC + cid
        pltpu.sync_copy(srci_hbm.at[wid], srci)
        pltpu.sync_copy(dsti_hbm.at[wid], dsti)
        pltpu.sync_copy(zrow_hbm, stage)
        base = sid * rows_pt
        for z in range(nzc):
            pltpu.sync_copy(stage, acc.at[pl.ds(base + z * C, C)])
        plsc.subcore_barrier()

        def body(j, carry):
            pltpu.sync_copy(g_hbm.at[srci.at[j]], msg)
            pltpu.sync_copy(msg, acc.at[dsti.at[j]], add=True)
            return carry

        lax.fori_loop(0, nchunk, body, 0)
        plsc.subcore_barrier()
        for z in range(nzc):
            sl = pl.ds(base + z * C, C)
            pltpu.sync_copy(acc.at[sl], stage)
            pltpu.sync_copy(stage, out_hbm.at[cid, sl])

    return scat


@functools.lru_cache(maxsize=None)
def _make_tc_first(n_pad: int, in_dim: int, hid: int):
    def body(degp_ref, x_ref, w_ref, g_ref, dinv_ref):
        deg = degp_ref[0, :, 0:1] + degp_ref[1, :, 0:1] + 1.0
        dinv = lax.rsqrt(deg)
        dinv_ref[...] = jnp.broadcast_to(dinv, (n_pad, 8))
        m = jnp.dot(x_ref[...], w_ref[...], preferred_element_type=jnp.float32)
        g_ref[...] = m * dinv

    return pl.pallas_call(
        body,
        out_shape=(
            jax.ShapeDtypeStruct((n_pad, hid), jnp.float32),
            jax.ShapeDtypeStruct((n_pad, 8), jnp.float32),
        ),
    )


@functools.lru_cache(maxsize=None)
def _make_tc_next(n_pad: int, d_in: int, d_out: int):
    def body(acc_ref, g_ref, dinv_ref, b_ref, w_ref, o_ref):
        dv = dinv_ref[:, 0:1]
        conv = dv * (acc_ref[0] + acc_ref[1] + g_ref[...]) + b_ref[...]
        h = jnp.maximum(conv, 0.0)
        o_ref[...] = dv * jnp.dot(h, w_ref[...],
                                  preferred_element_type=jnp.float32)

    return pl.pallas_call(
        body,
        out_shape=jax.ShapeDtypeStruct((n_pad, d_out), jnp.float32),
    )


@functools.lru_cache(maxsize=None)
def _make_tc_final(n_pad: int, d: int):
    def body(acc_ref, g_ref, dinv_ref, b_ref, o_ref):
        dv = dinv_ref[:, 0:1]
        o_ref[...] = dv * (acc_ref[0] + acc_ref[1] + g_ref[...]) + b_ref[...]

    return pl.pallas_call(
        body,
        out_shape=jax.ShapeDtypeStruct((n_pad, d), jnp.float32),
    )


def kernel(x, edge_index, W1, b1, W2, b2, W3, b3):
    n, in_dim = x.shape
    e = edge_index.shape[1]
    hid = W1.shape[1]
    emb = W3.shape[1]

    n_pad = -(-n // (NS * C)) * (NS * C)
    epw = -(-e // NW)
    nchunk = -(-epw // C)
    e_pad = NW * nchunk * C
    trash = jnp.int32(n)

    src = edge_index[0]
    dst = edge_index[1]
    pad = jnp.full((e_pad - e,), trash, dtype=jnp.int32)
    srcp = jnp.concatenate([src, pad]).reshape(NW, nchunk, C)
    dstp = jnp.concatenate([dst, pad]).reshape(NW, nchunk, C)

    x_pad = jnp.pad(x, ((0, n_pad - n), (0, 0)))
    ones16 = jnp.ones((n_pad, 16), dtype=jnp.float32)
    z16 = jnp.zeros((C, 16), dtype=jnp.float32)
    zh = jnp.zeros((C, hid), dtype=jnp.float32)
    emb_p = 16
    W3p = jnp.pad(W3, ((0, 0), (0, emb_p - emb)))
    b3p = jnp.pad(b3, (0, emb_p - emb)).reshape(1, emb_p)
    b1r = b1.reshape(1, hid)
    b2r = b2.reshape(1, hid)

    scat16 = _make_scatter(n_pad, nchunk, 16)
    scath = _make_scatter(n_pad, nchunk, hid)

    degp = scat16(ones16, srcp, dstp, z16)
    g1, dinv = _make_tc_first(n_pad, in_dim, hid)(degp, x_pad, W1)
    acc1 = scath(g1, srcp, dstp, zh)
    g2 = _make_tc_next(n_pad, hid, hid)(acc1, g1, dinv, b1r, W2)
    acc2 = scath(g2, srcp, dstp, zh)
    g3 = _make_tc_next(n_pad, hid, emb_p)(acc2, g2, dinv, b2r, W3p)
    acc3 = scat16(g3, srcp, dstp, z16)
    outp = _make_tc_final(n_pad, emb_p)(acc3, g3, dinv, b3p)
    return outp[:n, :emb]
